# baseline (speedup 1.0000x reference)
import dataclasses
import functools

import jax
import jax.numpy as jnp
from jax import lax
from jax.experimental import pallas as pl
from jax.experimental.pallas import tpu as pltpu
from jax.experimental.pallas import tpu_sc as plsc

BB, NQ, CC = 2, 50176, 96
HH, WW = 224, 224
NH, NP = 4, 4
HD = CC // NH
HDP = 32
R_TBL = BB * NH * HH * WW
PW = 4 * HDP

BLK = 1792
NQB = NQ // BLK
NBLK = BB * NQB
NTILES = 32
QPT = BB * NQ // NTILES
QB = 16
NIT = QPT // QB
NS = NH * NP
GW = 128
NG = QB * NS // GW


CA = CC + 2 + CC + 6


def _prework_body(qa_ref, wox_ref, box_ref, woy_ref, boy_ref,
                  wa_ref, ba_ref, wv_ref, bv_ref,
                  vp_ref, aux_ref):
    b = pl.program_id(0)
    qa = qa_ref[0]
    vp_ref[...] = jnp.dot(qa, wv_ref[...],
                          preferred_element_type=jnp.float32) + bv_ref[...]
    a = jnp.dot(qa, wa_ref[...], preferred_element_type=jnp.float32) + ba_ref[...]
    e = jnp.exp(a)
    col = lax.broadcasted_iota(jnp.int32, (NS, NS), 0) // NP
    row = lax.broadcasted_iota(jnp.int32, (NS, NS), 1) // NP
    g = (col == row).astype(jnp.float32)
    s = jnp.dot(e, g, preferred_element_type=jnp.float32)
    aw = e / s
    locx = jnp.clip(jnp.dot(qa, wox_ref[...], precision=lax.Precision.HIGHEST,
                            preferred_element_type=jnp.float32) + box_ref[...],
                    0.0, 1.0)
    locy = jnp.clip(jnp.dot(qa, woy_ref[...], precision=lax.Precision.HIGHEST,
                            preferred_element_type=jnp.float32) + boy_ref[...],
                    0.0, 1.0)
    ix = jnp.clip(locx * WW - 0.5, 0.0, WW - 1.0)
    iy = jnp.clip(locy * HH - 0.5, 0.0, HH - 1.0)
    x0 = jnp.floor(ix)
    y0 = jnp.floor(iy)
    wx1 = ix - x0
    wy1 = iy - y0
    wx0 = 1.0 - wx1
    wy0 = 1.0 - wy1
    wcat = jnp.concatenate(
        [aw * wy0 * wx0, aw * wy0 * wx1, aw * wy1 * wx0, aw * wy1 * wx1],
        axis=1)
    jo = lax.broadcasted_iota(jnp.int32, (4 * NS, 4 * NS), 0)
    jn = lax.broadcasted_iota(jnp.int32, (4 * NS, 4 * NS), 1)
    perm = ((jo % NS // NP) * 16 + (jo % NP) * 4 + jo // NS == jn)
    w4 = jnp.dot(wcat, perm.astype(jnp.float32),
                 preferred_element_type=jnp.float32)
    h = lax.broadcasted_iota(jnp.int32, (BLK, NS), 1) // NP
    r0 = (((b * NH + h) * HH + y0.astype(jnp.int32)) * WW
          + x0.astype(jnp.int32))
    aux_ref[...] = jnp.concatenate(
        [w4, lax.bitcast_convert_type(r0, jnp.float32),
         jnp.zeros((BLK, PW - 5 * NS), jnp.float32)], axis=1)


def _run_prework(qa3, wox, box, woy, boy, wa, ba, wv, bv):
    full = lambda s: pl.BlockSpec(s, lambda b, j: tuple(0 for _ in s))
    inblk = lambda n: pl.BlockSpec((1, BLK, n), lambda b, j: (b, j, 0))
    rowblk = lambda n: pl.BlockSpec((BLK, n), lambda b, j: (b * NQB + j, 0))
    return pl.pallas_call(
        _prework_body,
        grid=(BB, NQB),
        in_specs=[inblk(CA),
                  full((CA, NS)), full((NS,)), full((CA, NS)), full((NS,)),
                  full((CA, NS)), full((NS,)), full((CA, CC)), full((CC,))],
        out_specs=[rowblk(CC), rowblk(PW)],
        out_shape=[jax.ShapeDtypeStruct((BB * NQ, CC), jnp.float32),
                   jax.ShapeDtypeStruct((BB * NQ, PW), jnp.float32)],
    )(qa3, wox, box, woy, boy, wa, ba, wv, bv)


YB = 16
NYB = HH // YB
NSTEP = BB * NYB


def _table_body(vp_hbm, tbl_hbm, in_a, in_b, out_a, out_b, sem_in, sem_out):
    g = pl.program_id(0)
    in_bufs = (in_a, in_b)
    out_bufs = (out_a, out_b)

    def in_copies(s, buf):
        b = s // NYB
        yb = s % NYB
        row0 = b * NQ + yb * YB * WW
        start9 = jnp.where(yb == NYB - 1, row0 + (YB - 1) * WW,
                           row0 + YB * WW)
        return (pltpu.make_async_copy(vp_hbm.at[pl.ds(row0, YB * WW)],
                                      buf.at[pl.ds(0, YB * WW)], sem_in),
                pltpu.make_async_copy(vp_hbm.at[pl.ds(start9, WW)],
                                      buf.at[pl.ds(YB * WW, WW)], sem_in))

    def out_row(s, h):
        return ((s // NYB) * NH + h) * HH + (s % NYB) * YB

    @pl.when(g == 0)
    def _():
        for cp in in_copies(0, in_a):
            cp.start()

    def do_step(s, par):
        for cp in in_copies(s, in_bufs[par]):
            cp.wait()

        @pl.when(s + 1 < NSTEP)
        def _():
            for cp in in_copies(s + 1, in_bufs[1 - par]):
                cp.start()

        x = in_bufs[par][...].reshape(YB + 1, WW, CC)
        for h in range(NH):
            ob = out_bufs[h % 2]
            if h >= 2:
                pltpu.make_async_copy(
                    ob, tbl_hbm.at[pl.ds(out_row(s, h - 2), YB)],
                    sem_out).wait()
            else:
                @pl.when(s > 0)
                def _():
                    pltpu.make_async_copy(
                        ob, tbl_hbm.at[pl.ds(out_row(s - 1, h + 2), YB)],
                        sem_out).wait()
            c = x[:, :, h * HD:(h + 1) * HD]
            c = jnp.concatenate(
                [c, jnp.zeros((YB + 1, WW, HDP - HD), jnp.float32)], axis=2)
            r0 = c[0:YB]
            r1 = c[1:YB + 1]
            r0s = jnp.concatenate([r0[:, 1:WW, :], r0[:, WW - 1:WW, :]],
                                  axis=1)
            r1s = jnp.concatenate([r1[:, 1:WW, :], r1[:, WW - 1:WW, :]],
                                  axis=1)
            ob[...] = jnp.concatenate([r0, r0s, r1, r1s], axis=2)
            pltpu.make_async_copy(
                ob, tbl_hbm.at[pl.ds(out_row(s, h), YB)], sem_out).start()

    do_step(2 * g, 0)
    do_step(2 * g + 1, 1)

    @pl.when(g == NSTEP // 2 - 1)
    def _():
        for h in (2, 3):
            pltpu.make_async_copy(
                out_bufs[h % 2],
                tbl_hbm.at[pl.ds(out_row(NSTEP - 1, h), YB)],
                sem_out).wait()


def _run_table_build(vp):
    return pl.pallas_call(
        _table_body,
        grid=(NSTEP // 2,),
        in_specs=[pl.BlockSpec(memory_space=pl.ANY)],
        out_specs=pl.BlockSpec(memory_space=pl.ANY),
        out_shape=jax.ShapeDtypeStruct((BB * NH * HH, WW, PW), jnp.float32),
        scratch_shapes=[pltpu.VMEM(((YB + 1) * WW, CC), jnp.float32),
                        pltpu.VMEM(((YB + 1) * WW, CC), jnp.float32),
                        pltpu.VMEM((YB, WW, PW), jnp.float32),
                        pltpu.VMEM((YB, WW, PW), jnp.float32),
                        pltpu.SemaphoreType.DMA,
                        pltpu.SemaphoreType.DMA],
    )(vp)


def _sc_sample_combine(table, aux):
    mesh = plsc.VectorSubcoreMesh(core_axis_name="c", subcore_axis_name="s")
    cp = pltpu.CompilerParams()
    if "needs_layout_passes" in pltpu.CompilerParams.__dataclass_fields__:
        cp = dataclasses.replace(cp, needs_layout_passes=False)

    @functools.partial(
        pl.kernel, mesh=mesh, compiler_params=cp,
        out_type=jax.ShapeDtypeStruct((BB * NQ, NH * HDP), jnp.float32),
        scratch_types=[
            pltpu.VMEM((QB, PW), jnp.float32),
            pltpu.VMEM((QB, PW), jnp.float32),
            pltpu.VMEM((QB * NS,), jnp.int32),
            pltpu.VMEM((QB * NS,), jnp.int32),
            pltpu.VMEM((QB * NS, PW), jnp.float32),
            pltpu.VMEM((QB * NS, PW), jnp.float32),
            pltpu.VMEM((QB, NH * HDP), jnp.float32),
            pltpu.VMEM((QB, NH * HDP), jnp.float32),
            pltpu.SemaphoreType.DMA,
            pltpu.SemaphoreType.DMA,
            pltpu.SemaphoreType.DMA,
            pltpu.SemaphoreType.DMA,
        ])
    def sck(tbl_hbm, aux_hbm, out_hbm, aux_v0, aux_v1, idx_v0, idx_v1,
            patch_v0, patch_v1, out_v0, out_v1, gsem0, gsem1, osem0, osem1):
        wid = lax.axis_index("s") * 2 + lax.axis_index("c")
        base = wid * QPT
        aux_v = (aux_v0, aux_v1)
        idx_v = (idx_v0, idx_v1)
        patch_v = (patch_v0, patch_v1)
        out_v = (out_v0, out_v1)
        gsem = (gsem0, gsem1)
        osem = (osem0, osem1)

        def stage1(i, p):
            qb = base + i * QB
            pltpu.sync_copy(aux_hbm.at[pl.ds(qb, QB)], aux_v[p])
            for q in range(QB):
                iv = plsc.bitcast(aux_v[p][q, pl.ds(4 * NS, NS)], jnp.int32)
                idx_v[p][pl.ds(q * NS, NS)] = iv
            for gi in range(NG):
                pltpu.async_copy(
                    tbl_hbm.at[idx_v[p].at[pl.ds(gi * GW, GW)]],
                    patch_v[p].at[pl.ds(gi * GW, GW)], gsem[p])

        wcols = [jnp.full((16,), c, jnp.int32) for c in range(4 * NS)]

        def stage2(i, p, wait_out):
            qb = base + i * QB
            for gi in range(NG):
                pltpu.make_async_copy(
                    tbl_hbm.at[idx_v[p].at[pl.ds(gi * GW, GW)]],
                    patch_v[p].at[pl.ds(gi * GW, GW)], gsem[p]).wait()
            qb2 = base + (i - 2) * QB
            @pl.when(wait_out)
            def _():
                pltpu.make_async_copy(
                    out_v[p], out_hbm.at[pl.ds(qb2, QB)], osem[p]).wait()

            @pl.loop(0, QB)
            def _(q):
                qv = jnp.full((16,), q, jnp.int32)
                for h in range(NH):
                    acc = [jnp.zeros((16,), jnp.float32) for _ in range(2)]
                    for pp in range(NP):
                        s = q * NS + h * NP + pp
                        for ci in range(4):
                            wb = plsc.load_gather(
                                aux_v[p], [qv, wcols[h * 16 + pp * 4 + ci]])
                            for ch in range(HDP // 16):
                                acc[ch] = acc[ch] + wb * patch_v[p][
                                    s, pl.ds(ci * HDP + ch * 16, 16)]
                    for ch in range(HDP // 16):
                        out_v[p][q, pl.ds(h * HDP + ch * 16, 16)] = acc[ch]

            pltpu.async_copy(out_v[p], out_hbm.at[pl.ds(qb, QB)], osem[p])

        stage1(0, 0)

        @pl.loop(0, NIT, step=2)
        def _(k):
            stage1(k + 1, 1)
            stage2(k, 0, wait_out=k >= 2)

            @pl.when(k + 2 < NIT)
            def _():
                stage1(k + 2, 0)

            stage2(k + 1, 1, wait_out=k >= 2)

        for p in range(2):
            qbf = base + (NIT - 2 + p) * QB
            pltpu.make_async_copy(
                out_v[p], out_hbm.at[pl.ds(qbf, QB)], osem[p]).wait()

    return sck(table, aux)


def _proj_body(x_ref, w_ref, b_ref, o_ref):
    o_ref[0] = jnp.dot(x_ref[...], w_ref[...],
                       preferred_element_type=jnp.float32) + b_ref[...]


def _run_out_proj(samp, wo_pad, bo):
    full = lambda s: pl.BlockSpec(s, lambda b, j: tuple(0 for _ in s))
    return pl.pallas_call(
        _proj_body,
        grid=(BB, NQB),
        in_specs=[pl.BlockSpec((BLK, NH * HDP),
                               lambda b, j: (b * NQB + j, 0)),
                  full((NH * HDP, CC)), full((CC,))],
        out_specs=pl.BlockSpec((1, BLK, CC), lambda b, j: (b, j, 0)),
        out_shape=jax.ShapeDtypeStruct((BB, NQ, CC), jnp.float32),
    )(samp, wo_pad, bo)


def kernel(query, reference_points, value, spatial_shapes, W_off, b_off,
           W_attn, b_attn, W_v, b_v, W_o, b_o):
    del spatial_shapes
    qa3 = jnp.concatenate(
        [query, reference_points, value,
         jnp.zeros((BB, NQ, 6), jnp.float32)], axis=-1)
    z2 = jnp.zeros((2, NS), jnp.float32)
    zc = jnp.zeros((CC, NS), jnp.float32)
    z6 = jnp.zeros((6, NS), jnp.float32)
    ex = jnp.zeros((2, NS), jnp.float32).at[0, :].set(1.0)
    ey = jnp.zeros((2, NS), jnp.float32).at[1, :].set(1.0)
    wox = jnp.concatenate([W_off[:, 0::2] * (0.1 / WW), ex, zc, z6], axis=0)
    woy = jnp.concatenate([W_off[:, 1::2] * (0.1 / HH), ey, zc, z6], axis=0)
    box = b_off[0::2] * (0.1 / WW)
    boy = b_off[1::2] * (0.1 / HH)
    wa = jnp.concatenate([W_attn, z2, zc, z6], axis=0)
    wv = jnp.concatenate([jnp.zeros((CC + 2, CC), jnp.float32), W_v,
                          jnp.zeros((6, CC), jnp.float32)], axis=0)

    vp, aux = _run_prework(qa3, wox, box, woy, boy, wa, b_attn, wv, b_v)

    table = _run_table_build(vp).reshape(R_TBL, PW)

    samp = _sc_sample_combine(table, aux)

    wo = W_o.reshape(NH, HD, CC)
    wo_pad = jnp.pad(wo, ((0, 0), (0, HDP - HD), (0, 0))).reshape(NH * HDP, CC)
    return _run_out_proj(samp, wo_pad, b_o)


# speedup vs baseline: 1.1695x; 1.1695x over previous
import dataclasses
import functools

import jax
import jax.numpy as jnp
from jax import lax
from jax.experimental import pallas as pl
from jax.experimental.pallas import tpu as pltpu
from jax.experimental.pallas import tpu_sc as plsc

BB, NQ, CC = 2, 50176, 96
HH, WW = 224, 224
NH, NP = 4, 4
HD = CC // NH
HDP = 32
R_TBL = BB * NH * HH * WW
PW = 4 * HDP

BLK = 1792
NQB = NQ // BLK
NBLK = BB * NQB
NTILES = 32
QPT = BB * NQ // NTILES
QB = 16
NIT = QPT // QB
NS = NH * NP
GW = 128
NG = QB * NS // GW


def _prework_body(q_ref, rp_ref, v_ref, wox_ref, box_ref, woy_ref, boy_ref,
                  wa_ref, ba_ref, wv_ref, bv_ref,
                  vp_ref, aux_ref):
    b = pl.program_id(0)
    q = q_ref[0]
    vp_ref[...] = jnp.dot(v_ref[0], wv_ref[...],
                          preferred_element_type=jnp.float32) + bv_ref[...]
    a = jnp.dot(q, wa_ref[...], preferred_element_type=jnp.float32) + ba_ref[...]
    e = jnp.exp(a)
    col = lax.broadcasted_iota(jnp.int32, (NS, NS), 0) // NP
    row = lax.broadcasted_iota(jnp.int32, (NS, NS), 1) // NP
    g = (col == row).astype(jnp.float32)
    s = jnp.dot(e, g, preferred_element_type=jnp.float32)
    aw = e / s
    offx = jnp.dot(q, wox_ref[...], preferred_element_type=jnp.float32) + box_ref[...]
    offy = jnp.dot(q, woy_ref[...], preferred_element_type=jnp.float32) + boy_ref[...]
    rx = rp_ref[0, :, 0:1]
    ry = rp_ref[0, :, 1:2]
    locx = jnp.clip(rx + offx * (0.1 / WW), 0.0, 1.0)
    locy = jnp.clip(ry + offy * (0.1 / HH), 0.0, 1.0)
    ix = jnp.clip(locx * WW - 0.5, 0.0, WW - 1.0)
    iy = jnp.clip(locy * HH - 0.5, 0.0, HH - 1.0)
    x0 = jnp.floor(ix)
    y0 = jnp.floor(iy)
    wx1 = ix - x0
    wy1 = iy - y0
    wx0 = 1.0 - wx1
    wy0 = 1.0 - wy1
    wcat = jnp.concatenate(
        [aw * wy0 * wx0, aw * wy0 * wx1, aw * wy1 * wx0, aw * wy1 * wx1],
        axis=1)
    jo = lax.broadcasted_iota(jnp.int32, (4 * NS, 4 * NS), 0)
    jn = lax.broadcasted_iota(jnp.int32, (4 * NS, 4 * NS), 1)
    perm = ((jo % NS // NP) * 16 + (jo % NP) * 4 + jo // NS == jn)
    w4 = jnp.dot(wcat, perm.astype(jnp.float32),
                 preferred_element_type=jnp.float32)
    h = lax.broadcasted_iota(jnp.int32, (BLK, NS), 1) // NP
    r0 = (((b * NH + h) * HH + y0.astype(jnp.int32)) * WW
          + x0.astype(jnp.int32))
    aux_ref[...] = jnp.concatenate(
        [w4, lax.bitcast_convert_type(r0, jnp.float32),
         jnp.zeros((BLK, PW - 5 * NS), jnp.float32)], axis=1)


def _run_prework(query3, rp3, value3, wox, box, woy, boy, wa, ba, wv, bv):
    full = lambda s: pl.BlockSpec(s, lambda b, j: tuple(0 for _ in s))
    inblk = lambda n: pl.BlockSpec((1, BLK, n), lambda b, j: (b, j, 0))
    rowblk = lambda n: pl.BlockSpec((BLK, n), lambda b, j: (b * NQB + j, 0))
    return pl.pallas_call(
        _prework_body,
        grid=(BB, NQB),
        in_specs=[inblk(CC), inblk(2), inblk(CC),
                  full((CC, NS)), full((NS,)), full((CC, NS)), full((NS,)),
                  full((CC, NS)), full((NS,)), full((CC, CC)), full((CC,))],
        out_specs=[rowblk(CC), rowblk(PW)],
        out_shape=[jax.ShapeDtypeStruct((BB * NQ, CC), jnp.float32),
                   jax.ShapeDtypeStruct((BB * NQ, PW), jnp.float32)],
    )(query3, rp3, value3, wox, box, woy, boy, wa, ba, wv, bv)


YB = 16
NYB = HH // YB
NSTEP = BB * NYB


def _table_body(vp_hbm, tbl_hbm, in_a, in_b, out_a, out_b, sem_in, sem_out):
    g = pl.program_id(0)
    in_bufs = (in_a, in_b)
    out_bufs = (out_a, out_b)

    def in_copies(s, buf):
        b = s // NYB
        yb = s % NYB
        row0 = b * NQ + yb * YB * WW
        start9 = jnp.where(yb == NYB - 1, row0 + (YB - 1) * WW,
                           row0 + YB * WW)
        return (pltpu.make_async_copy(vp_hbm.at[pl.ds(row0, YB * WW)],
                                      buf.at[pl.ds(0, YB * WW)], sem_in),
                pltpu.make_async_copy(vp_hbm.at[pl.ds(start9, WW)],
                                      buf.at[pl.ds(YB * WW, WW)], sem_in))

    def out_row(s, h):
        return ((s // NYB) * NH + h) * HH + (s % NYB) * YB

    @pl.when(g == 0)
    def _():
        for cp in in_copies(0, in_a):
            cp.start()

    def do_step(s, par):
        for cp in in_copies(s, in_bufs[par]):
            cp.wait()

        @pl.when(s + 1 < NSTEP)
        def _():
            for cp in in_copies(s + 1, in_bufs[1 - par]):
                cp.start()

        x = in_bufs[par][...].reshape(YB + 1, WW, CC)
        for h in range(NH):
            ob = out_bufs[h % 2]
            if h >= 2:
                pltpu.make_async_copy(
                    ob, tbl_hbm.at[pl.ds(out_row(s, h - 2), YB)],
                    sem_out).wait()
            else:
                @pl.when(s > 0)
                def _():
                    pltpu.make_async_copy(
                        ob, tbl_hbm.at[pl.ds(out_row(s - 1, h + 2), YB)],
                        sem_out).wait()
            c = x[:, :, h * HD:(h + 1) * HD]
            c = jnp.concatenate(
                [c, jnp.zeros((YB + 1, WW, HDP - HD), jnp.float32)], axis=2)
            r0 = c[0:YB]
            r1 = c[1:YB + 1]
            r0s = jnp.concatenate([r0[:, 1:WW, :], r0[:, WW - 1:WW, :]],
                                  axis=1)
            r1s = jnp.concatenate([r1[:, 1:WW, :], r1[:, WW - 1:WW, :]],
                                  axis=1)
            ob[...] = jnp.concatenate([r0, r0s, r1, r1s], axis=2)
            pltpu.make_async_copy(
                ob, tbl_hbm.at[pl.ds(out_row(s, h), YB)], sem_out).start()

    do_step(2 * g, 0)
    do_step(2 * g + 1, 1)

    @pl.when(g == NSTEP // 2 - 1)
    def _():
        for h in (2, 3):
            pltpu.make_async_copy(
                out_bufs[h % 2],
                tbl_hbm.at[pl.ds(out_row(NSTEP - 1, h), YB)],
                sem_out).wait()


def _run_table_build(vp):
    return pl.pallas_call(
        _table_body,
        grid=(NSTEP // 2,),
        in_specs=[pl.BlockSpec(memory_space=pl.ANY)],
        out_specs=pl.BlockSpec(memory_space=pl.ANY),
        out_shape=jax.ShapeDtypeStruct((BB * NH * HH, WW, PW), jnp.float32),
        scratch_shapes=[pltpu.VMEM(((YB + 1) * WW, CC), jnp.float32),
                        pltpu.VMEM(((YB + 1) * WW, CC), jnp.float32),
                        pltpu.VMEM((YB, WW, PW), jnp.float32),
                        pltpu.VMEM((YB, WW, PW), jnp.float32),
                        pltpu.SemaphoreType.DMA,
                        pltpu.SemaphoreType.DMA],
    )(vp)


def _sc_sample_combine(table, aux):
    mesh = plsc.VectorSubcoreMesh(core_axis_name="c", subcore_axis_name="s")
    cp = pltpu.CompilerParams()
    if "needs_layout_passes" in pltpu.CompilerParams.__dataclass_fields__:
        cp = dataclasses.replace(cp, needs_layout_passes=False)

    @functools.partial(
        pl.kernel, mesh=mesh, compiler_params=cp,
        out_type=jax.ShapeDtypeStruct((BB * NQ, NH * HDP), jnp.float32),
        scratch_types=[
            pltpu.VMEM((QB, PW), jnp.float32),
            pltpu.VMEM((QB, PW), jnp.float32),
            pltpu.VMEM((QB * NS,), jnp.int32),
            pltpu.VMEM((QB * NS,), jnp.int32),
            pltpu.VMEM((QB * NS, PW), jnp.float32),
            pltpu.VMEM((QB * NS, PW), jnp.float32),
            pltpu.VMEM((QB, NH * HDP), jnp.float32),
            pltpu.VMEM((QB, NH * HDP), jnp.float32),
            pltpu.SemaphoreType.DMA,
            pltpu.SemaphoreType.DMA,
            pltpu.SemaphoreType.DMA,
            pltpu.SemaphoreType.DMA,
        ])
    def sck(tbl_hbm, aux_hbm, out_hbm, aux_v0, aux_v1, idx_v0, idx_v1,
            patch_v0, patch_v1, out_v0, out_v1, gsem0, gsem1, osem0, osem1):
        wid = lax.axis_index("s") * 2 + lax.axis_index("c")
        base = wid * QPT
        aux_v = (aux_v0, aux_v1)
        idx_v = (idx_v0, idx_v1)
        patch_v = (patch_v0, patch_v1)
        out_v = (out_v0, out_v1)
        gsem = (gsem0, gsem1)
        osem = (osem0, osem1)

        def stage1(i, p):
            qb = base + i * QB
            pltpu.sync_copy(aux_hbm.at[pl.ds(qb, QB)], aux_v[p])
            for q in range(QB):
                iv = plsc.bitcast(aux_v[p][q, pl.ds(4 * NS, NS)], jnp.int32)
                idx_v[p][pl.ds(q * NS, NS)] = iv
            for gi in range(NG):
                pltpu.async_copy(
                    tbl_hbm.at[idx_v[p].at[pl.ds(gi * GW, GW)]],
                    patch_v[p].at[pl.ds(gi * GW, GW)], gsem[p])

        wcols = [jnp.full((16,), c, jnp.int32) for c in range(4 * NS)]

        def stage2(i, p, wait_out):
            qb = base + i * QB
            for gi in range(NG):
                pltpu.make_async_copy(
                    tbl_hbm.at[idx_v[p].at[pl.ds(gi * GW, GW)]],
                    patch_v[p].at[pl.ds(gi * GW, GW)], gsem[p]).wait()
            qb2 = base + (i - 2) * QB
            @pl.when(wait_out)
            def _():
                pltpu.make_async_copy(
                    out_v[p], out_hbm.at[pl.ds(qb2, QB)], osem[p]).wait()

            @pl.loop(0, QB)
            def _(q):
                qv = jnp.full((16,), q, jnp.int32)
                for h in range(NH):
                    acc = [jnp.zeros((16,), jnp.float32) for _ in range(2)]
                    for pp in range(NP):
                        s = q * NS + h * NP + pp
                        for ci in range(4):
                            wb = plsc.load_gather(
                                aux_v[p], [qv, wcols[h * 16 + pp * 4 + ci]])
                            for ch in range(HDP // 16):
                                acc[ch] = acc[ch] + wb * patch_v[p][
                                    s, pl.ds(ci * HDP + ch * 16, 16)]
                    for ch in range(HDP // 16):
                        out_v[p][q, pl.ds(h * HDP + ch * 16, 16)] = acc[ch]

            pltpu.async_copy(out_v[p], out_hbm.at[pl.ds(qb, QB)], osem[p])

        stage1(0, 0)

        @pl.loop(0, NIT, step=2)
        def _(k):
            stage1(k + 1, 1)
            stage2(k, 0, wait_out=k >= 2)

            @pl.when(k + 2 < NIT)
            def _():
                stage1(k + 2, 0)

            stage2(k + 1, 1, wait_out=k >= 2)

        for p in range(2):
            qbf = base + (NIT - 2 + p) * QB
            pltpu.make_async_copy(
                out_v[p], out_hbm.at[pl.ds(qbf, QB)], osem[p]).wait()

    return sck(table, aux)


def _proj_body(x_ref, w_ref, b_ref, o_ref):
    o_ref[0] = jnp.dot(x_ref[...], w_ref[...],
                       preferred_element_type=jnp.float32) + b_ref[...]


def _run_out_proj(samp, wo_pad, bo):
    full = lambda s: pl.BlockSpec(s, lambda b, j: tuple(0 for _ in s))
    return pl.pallas_call(
        _proj_body,
        grid=(BB, NQB),
        in_specs=[pl.BlockSpec((BLK, NH * HDP),
                               lambda b, j: (b * NQB + j, 0)),
                  full((NH * HDP, CC)), full((CC,))],
        out_specs=pl.BlockSpec((1, BLK, CC), lambda b, j: (b, j, 0)),
        out_shape=jax.ShapeDtypeStruct((BB, NQ, CC), jnp.float32),
    )(samp, wo_pad, bo)


def kernel(query, reference_points, value, spatial_shapes, W_off, b_off,
           W_attn, b_attn, W_v, b_v, W_o, b_o):
    del spatial_shapes
    wox, woy = W_off[:, 0::2], W_off[:, 1::2]
    box, boy = b_off[0::2], b_off[1::2]

    vp, aux = _run_prework(query, reference_points, value, wox, box, woy,
                           boy, W_attn, b_attn, W_v, b_v)

    table = _run_table_build(vp).reshape(R_TBL, PW)

    samp = _sc_sample_combine(table, aux)

    wo = W_o.reshape(NH, HD, CC)
    wo_pad = jnp.pad(wo, ((0, 0), (0, HDP - HD), (0, 0))).reshape(NH * HDP, CC)
    return _run_out_proj(samp, wo_pad, b_o)


# speedup vs baseline: 1.2483x; 1.0674x over previous
import dataclasses
import functools

import jax
import jax.numpy as jnp
from jax import lax
from jax.experimental import pallas as pl
from jax.experimental.pallas import tpu as pltpu
from jax.experimental.pallas import tpu_sc as plsc

BB, NQ, CC = 2, 50176, 96
HH, WW = 224, 224
NH, NP = 4, 4
HD = CC // NH
HDP = 32
R_TBL = BB * NH * HH * WW
PW = 4 * HDP

BLK = 1792
NQB = NQ // BLK
NBLK = BB * NQB
NTILES = 32
QPT = BB * NQ // NTILES
QB = 16
NIT = QPT // QB
NS = NH * NP
GW = 128
NG = QB * NS // GW


def _prework_body(q_ref, rp_ref, v_ref, wox_ref, box_ref, woy_ref, boy_ref,
                  wa_ref, ba_ref, wv_ref, bv_ref,
                  vp_ref, aux_ref):
    b = pl.program_id(0)
    q = q_ref[0]
    vp_ref[...] = jnp.dot(v_ref[0], wv_ref[...],
                          preferred_element_type=jnp.float32) + bv_ref[...]
    a = jnp.dot(q, wa_ref[...], preferred_element_type=jnp.float32) + ba_ref[...]
    e = jnp.exp(a)
    col = lax.broadcasted_iota(jnp.int32, (NS, NS), 0) // NP
    row = lax.broadcasted_iota(jnp.int32, (NS, NS), 1) // NP
    g = (col == row).astype(jnp.float32)
    s = jnp.dot(e, g, preferred_element_type=jnp.float32)
    aw = e / s
    offx = jnp.dot(q, wox_ref[...], preferred_element_type=jnp.float32) + box_ref[...]
    offy = jnp.dot(q, woy_ref[...], preferred_element_type=jnp.float32) + boy_ref[...]
    rx = rp_ref[0, :, 0:1]
    ry = rp_ref[0, :, 1:2]
    locx = jnp.clip(rx + offx * (0.1 / WW), 0.0, 1.0)
    locy = jnp.clip(ry + offy * (0.1 / HH), 0.0, 1.0)
    ix = jnp.clip(locx * WW - 0.5, 0.0, WW - 1.0)
    iy = jnp.clip(locy * HH - 0.5, 0.0, HH - 1.0)
    x0 = jnp.floor(ix)
    y0 = jnp.floor(iy)
    wx1 = ix - x0
    wy1 = iy - y0
    wx0 = 1.0 - wx1
    wy0 = 1.0 - wy1
    wcat = jnp.concatenate(
        [aw * wy0 * wx0, aw * wy0 * wx1, aw * wy1 * wx0, aw * wy1 * wx1],
        axis=1)
    jo = lax.broadcasted_iota(jnp.int32, (4 * NS, 4 * NS), 0)
    jn = lax.broadcasted_iota(jnp.int32, (4 * NS, 4 * NS), 1)
    perm = ((jo % NS // NP) * 16 + (jo % NP) * 4 + jo // NS == jn)
    w4 = jnp.dot(wcat, perm.astype(jnp.float32),
                 preferred_element_type=jnp.float32)
    h = lax.broadcasted_iota(jnp.int32, (BLK, NS), 1) // NP
    r0 = (((b * NH + h) * HH + y0.astype(jnp.int32)) * WW
          + x0.astype(jnp.int32))
    aux_ref[...] = jnp.concatenate(
        [w4, lax.bitcast_convert_type(r0, jnp.float32),
         jnp.zeros((BLK, PW - 5 * NS), jnp.float32)], axis=1)


def _run_prework(query3, rp3, value3, wox, box, woy, boy, wa, ba, wv, bv):
    full = lambda s: pl.BlockSpec(s, lambda b, j: tuple(0 for _ in s))
    inblk = lambda n: pl.BlockSpec((1, BLK, n), lambda b, j: (b, j, 0))
    rowblk = lambda n: pl.BlockSpec((BLK, n), lambda b, j: (b * NQB + j, 0))
    return pl.pallas_call(
        _prework_body,
        grid=(BB, NQB),
        in_specs=[inblk(CC), inblk(2), inblk(CC),
                  full((CC, NS)), full((NS,)), full((CC, NS)), full((NS,)),
                  full((CC, NS)), full((NS,)), full((CC, PW)), full((PW,))],
        out_specs=[rowblk(PW), rowblk(PW)],
        out_shape=[jax.ShapeDtypeStruct((BB * NQ, PW), jnp.float32),
                   jax.ShapeDtypeStruct((BB * NQ, PW), jnp.float32)],
    )(query3, rp3, value3, wox, box, woy, boy, wa, ba, wv, bv)


YB = 16
NYB = HH // YB
NSTEP = BB * NYB


def _table_body(vp_hbm, tbl_hbm, in_a, in_b, out_a, out_b, sem_in, sem_out):
    g = pl.program_id(0)
    in_bufs = (in_a, in_b)
    out_bufs = (out_a, out_b)

    def in_copies(s, buf):
        b = s // NYB
        yb = s % NYB
        row0 = b * NQ + yb * YB * WW
        start9 = jnp.where(yb == NYB - 1, row0 + (YB - 1) * WW,
                           row0 + YB * WW)
        return (pltpu.make_async_copy(vp_hbm.at[pl.ds(row0, YB * WW)],
                                      buf.at[pl.ds(0, YB * WW)], sem_in),
                pltpu.make_async_copy(vp_hbm.at[pl.ds(start9, WW)],
                                      buf.at[pl.ds(YB * WW, WW)], sem_in))

    def out_row(s, h):
        return ((s // NYB) * NH + h) * HH + (s % NYB) * YB

    @pl.when(g == 0)
    def _():
        for cp in in_copies(0, in_a):
            cp.start()

    def do_step(s, par):
        for cp in in_copies(s, in_bufs[par]):
            cp.wait()

        @pl.when(s + 1 < NSTEP)
        def _():
            for cp in in_copies(s + 1, in_bufs[1 - par]):
                cp.start()

        x = in_bufs[par][...].reshape(YB + 1, WW, PW)
        for h in range(NH):
            ob = out_bufs[h % 2]
            if h >= 2:
                pltpu.make_async_copy(
                    ob, tbl_hbm.at[pl.ds(out_row(s, h - 2), YB)],
                    sem_out).wait()
            else:
                @pl.when(s > 0)
                def _():
                    pltpu.make_async_copy(
                        ob, tbl_hbm.at[pl.ds(out_row(s - 1, h + 2), YB)],
                        sem_out).wait()
            c = x[:, :, h * HDP:(h + 1) * HDP]
            r0 = c[0:YB]
            r1 = c[1:YB + 1]
            r0s = jnp.concatenate([r0[:, 1:WW, :], r0[:, WW - 1:WW, :]],
                                  axis=1)
            r1s = jnp.concatenate([r1[:, 1:WW, :], r1[:, WW - 1:WW, :]],
                                  axis=1)
            ob[...] = jnp.concatenate([r0, r0s, r1, r1s], axis=2)
            pltpu.make_async_copy(
                ob, tbl_hbm.at[pl.ds(out_row(s, h), YB)], sem_out).start()

    do_step(2 * g, 0)
    do_step(2 * g + 1, 1)

    @pl.when(g == NSTEP // 2 - 1)
    def _():
        for h in (2, 3):
            pltpu.make_async_copy(
                out_bufs[h % 2],
                tbl_hbm.at[pl.ds(out_row(NSTEP - 1, h), YB)],
                sem_out).wait()


def _run_table_build(vp):
    return pl.pallas_call(
        _table_body,
        grid=(NSTEP // 2,),
        in_specs=[pl.BlockSpec(memory_space=pl.ANY)],
        out_specs=pl.BlockSpec(memory_space=pl.ANY),
        out_shape=jax.ShapeDtypeStruct((BB * NH * HH, WW, PW), jnp.float32),
        scratch_shapes=[pltpu.VMEM(((YB + 1) * WW, PW), jnp.float32),
                        pltpu.VMEM(((YB + 1) * WW, PW), jnp.float32),
                        pltpu.VMEM((YB, WW, PW), jnp.float32),
                        pltpu.VMEM((YB, WW, PW), jnp.float32),
                        pltpu.SemaphoreType.DMA,
                        pltpu.SemaphoreType.DMA],
    )(vp)


def _sc_sample_combine(table, aux):
    mesh = plsc.VectorSubcoreMesh(core_axis_name="c", subcore_axis_name="s")
    cp = pltpu.CompilerParams()
    if "needs_layout_passes" in pltpu.CompilerParams.__dataclass_fields__:
        cp = dataclasses.replace(cp, needs_layout_passes=False)

    @functools.partial(
        pl.kernel, mesh=mesh, compiler_params=cp,
        out_type=jax.ShapeDtypeStruct((BB * NQ, NH * HDP), jnp.float32),
        scratch_types=[
            pltpu.VMEM((QB, PW), jnp.float32),
            pltpu.VMEM((QB, PW), jnp.float32),
            pltpu.VMEM((QB * NS,), jnp.int32),
            pltpu.VMEM((QB * NS,), jnp.int32),
            pltpu.VMEM((QB * NS, PW), jnp.float32),
            pltpu.VMEM((QB * NS, PW), jnp.float32),
            pltpu.VMEM((QB, NH * HDP), jnp.float32),
            pltpu.VMEM((QB, NH * HDP), jnp.float32),
            pltpu.SemaphoreType.DMA,
            pltpu.SemaphoreType.DMA,
            pltpu.SemaphoreType.DMA,
            pltpu.SemaphoreType.DMA,
        ])
    def sck(tbl_hbm, aux_hbm, out_hbm, aux_v0, aux_v1, idx_v0, idx_v1,
            patch_v0, patch_v1, out_v0, out_v1, gsem0, gsem1, osem0, osem1):
        wid = lax.axis_index("s") * 2 + lax.axis_index("c")
        base = wid * QPT
        aux_v = (aux_v0, aux_v1)
        idx_v = (idx_v0, idx_v1)
        patch_v = (patch_v0, patch_v1)
        out_v = (out_v0, out_v1)
        gsem = (gsem0, gsem1)
        osem = (osem0, osem1)

        def stage1(i, p):
            qb = base + i * QB
            pltpu.sync_copy(aux_hbm.at[pl.ds(qb, QB)], aux_v[p])
            for q in range(QB):
                iv = plsc.bitcast(aux_v[p][q, pl.ds(4 * NS, NS)], jnp.int32)
                idx_v[p][pl.ds(q * NS, NS)] = iv
            for gi in range(NG):
                pltpu.async_copy(
                    tbl_hbm.at[idx_v[p].at[pl.ds(gi * GW, GW)]],
                    patch_v[p].at[pl.ds(gi * GW, GW)], gsem[p])

        wcols = [jnp.full((16,), c, jnp.int32) for c in range(4 * NS)]

        def stage2(i, p, wait_out):
            qb = base + i * QB
            for gi in range(NG):
                pltpu.make_async_copy(
                    tbl_hbm.at[idx_v[p].at[pl.ds(gi * GW, GW)]],
                    patch_v[p].at[pl.ds(gi * GW, GW)], gsem[p]).wait()
            qb2 = base + (i - 2) * QB
            @pl.when(wait_out)
            def _():
                pltpu.make_async_copy(
                    out_v[p], out_hbm.at[pl.ds(qb2, QB)], osem[p]).wait()

            @pl.loop(0, QB)
            def _(q):
                qv = jnp.full((16,), q, jnp.int32)
                for h in range(NH):
                    acc = [jnp.zeros((16,), jnp.float32) for _ in range(2)]
                    for pp in range(NP):
                        s = q * NS + h * NP + pp
                        for ci in range(4):
                            wb = plsc.load_gather(
                                aux_v[p], [qv, wcols[h * 16 + pp * 4 + ci]])
                            for ch in range(HDP // 16):
                                acc[ch] = acc[ch] + wb * patch_v[p][
                                    s, pl.ds(ci * HDP + ch * 16, 16)]
                    for ch in range(HDP // 16):
                        out_v[p][q, pl.ds(h * HDP + ch * 16, 16)] = acc[ch]

            pltpu.async_copy(out_v[p], out_hbm.at[pl.ds(qb, QB)], osem[p])

        stage1(0, 0)

        @pl.loop(0, NIT, step=2)
        def _(k):
            stage1(k + 1, 1)
            stage2(k, 0, wait_out=k >= 2)

            @pl.when(k + 2 < NIT)
            def _():
                stage1(k + 2, 0)

            stage2(k + 1, 1, wait_out=k >= 2)

        for p in range(2):
            qbf = base + (NIT - 2 + p) * QB
            pltpu.make_async_copy(
                out_v[p], out_hbm.at[pl.ds(qbf, QB)], osem[p]).wait()

    return sck(table, aux)


def _proj_body(x_ref, w_ref, b_ref, o_ref):
    o_ref[0] = jnp.dot(x_ref[...], w_ref[...],
                       preferred_element_type=jnp.float32) + b_ref[...]


def _run_out_proj(samp, wo_pad, bo):
    full = lambda s: pl.BlockSpec(s, lambda b, j: tuple(0 for _ in s))
    return pl.pallas_call(
        _proj_body,
        grid=(BB, NQB),
        in_specs=[pl.BlockSpec((BLK, NH * HDP),
                               lambda b, j: (b * NQB + j, 0)),
                  full((NH * HDP, CC)), full((CC,))],
        out_specs=pl.BlockSpec((1, BLK, CC), lambda b, j: (b, j, 0)),
        out_shape=jax.ShapeDtypeStruct((BB, NQ, CC), jnp.float32),
    )(samp, wo_pad, bo)


def kernel(query, reference_points, value, spatial_shapes, W_off, b_off,
           W_attn, b_attn, W_v, b_v, W_o, b_o):
    del spatial_shapes
    wox, woy = W_off[:, 0::2], W_off[:, 1::2]
    box, boy = b_off[0::2], b_off[1::2]

    wv_pad = jnp.pad(W_v.reshape(CC, NH, HD),
                     ((0, 0), (0, 0), (0, HDP - HD))).reshape(CC, PW)
    bv_pad = jnp.pad(b_v.reshape(NH, HD), ((0, 0), (0, HDP - HD))).reshape(PW)

    vp, aux = _run_prework(query, reference_points, value, wox, box, woy,
                           boy, W_attn, b_attn, wv_pad, bv_pad)

    table = _run_table_build(vp).reshape(R_TBL, PW)

    samp = _sc_sample_combine(table, aux)

    wo = W_o.reshape(NH, HD, CC)
    wo_pad = jnp.pad(wo, ((0, 0), (0, HDP - HD), (0, 0))).reshape(NH * HDP, CC)
    return _run_out_proj(samp, wo_pad, b_o)
